# baseline (speedup 1.0000x reference)
import functools

import jax
import jax.numpy as jnp
from jax import lax
from jax.experimental import pallas as pl
from jax.experimental.pallas import tpu as pltpu
from jax.experimental.pallas import tpu_sc as plsc

B = 4096
L = 50
EMB = 64
VOCAB = 100000
NC = 2
NS = 16
NW = NC * NS
BPW = B // NW
LANES = 16
NBUF = 4

CV = 128
NCHUNK = VOCAB // CV
TAIL = VOCAB - NCHUNK * CV
TAILW = NCHUNK % NW
U32C = EMB // 2
CHUNK_OUT = CV * U32C


def _tp_kernel(tt_hbm, out_hbm, in_bufs, tail_buf, out_bufs, in_sems,
               out_sems):
    wid = lax.axis_index("s") * NC + lax.axis_index("c")
    iota = lax.iota(jnp.int32, LANES)
    rows = [iota + 16 * g for g in range(4)]

    def start_in(i, k):
        c = wid + NW * i

        @pl.when(c < NCHUNK)
        def _():
            off = pl.multiple_of(c * CV, CV)
            pltpu.async_copy(tt_hbm.at[:, pl.ds(off, CV)], in_bufs[k],
                             in_sems[k])

    def transpose_block(src, nvoc, out_buf):
        def vbody(vv, carry):
            for u in range(4):
                v = vv * 4 + u
                col = jnp.full((LANES,), 0, jnp.int32) + v
                g = [plsc.load_gather(src, [r, col]) for r in rows]
                p0 = plsc.bitcast(
                    plsc.pack(g[0], g[1], format=plsc.PackFormat.INTERLEAVED),
                    jnp.uint32)
                p1 = plsc.bitcast(
                    plsc.pack(g[2], g[3], format=plsc.PackFormat.INTERLEAVED),
                    jnp.uint32)
                out_buf[pl.ds(pl.multiple_of(v * U32C, 16), LANES)] = p0
                out_buf[pl.ds(pl.multiple_of(v * U32C + 16, 16), LANES)] = p1
            return carry

        lax.fori_loop(0, nvoc // 4, vbody, 0)

    for k in range(2):
        start_in(k, k)

    def body(g, carry):
        for k in range(2):
            i = 2 * g + k
            c = wid + NW * i

            @pl.when(c < NCHUNK)
            def _():
                off = pl.multiple_of(c * CV, CV)
                pltpu.make_async_copy(tt_hbm.at[:, pl.ds(off, CV)],
                                      in_bufs[k], in_sems[k]).wait()

                @pl.when(i >= 2)
                def _():
                    pltpu.make_async_copy(
                        out_bufs[k], out_hbm.at[pl.ds(0, CHUNK_OUT)],
                        out_sems[k]).wait()

                transpose_block(in_bufs[k], CV, out_bufs[k])
                pltpu.async_copy(
                    out_bufs[k],
                    out_hbm.at[pl.ds(pl.multiple_of(c * CHUNK_OUT, 8),
                                     CHUNK_OUT)],
                    out_sems[k])
                start_in(i + 2, k)
        return carry

    niter = (NCHUNK + NW - 1) // NW
    lax.fori_loop(0, (niter + 1) // 2, body, 0)

    for k in range(2):
        pltpu.make_async_copy(out_bufs[k], out_hbm.at[pl.ds(0, CHUNK_OUT)],
                              out_sems[k]).wait()

    @pl.when(wid == TAILW)
    def _():
        pltpu.sync_copy(tt_hbm.at[:, pl.ds(NCHUNK * CV, TAIL)], tail_buf)
        transpose_block(tail_buf, TAIL, out_bufs[0])
        pltpu.sync_copy(out_bufs[0].at[pl.ds(0, TAIL * U32C)],
                        out_hbm.at[pl.ds(NCHUNK * CHUNK_OUT, TAIL * U32C)])


@functools.partial(
    pl.kernel,
    mesh=plsc.VectorSubcoreMesh(core_axis_name="c", subcore_axis_name="s"),
    out_type=jax.ShapeDtypeStruct((VOCAB * U32C,), jnp.uint32),
    scratch_types=[
        *[pltpu.VMEM((EMB, CV), jnp.float32) for _ in range(2)],
        pltpu.VMEM((EMB, TAIL), jnp.float32),
        *[pltpu.VMEM((CHUNK_OUT,), jnp.uint32) for _ in range(2)],
        *[pltpu.SemaphoreType.DMA for _ in range(4)],
    ],
    compiler_params=pltpu.CompilerParams(use_tc_tiling_on_sc=True,
                                         needs_layout_passes=False),
)
def _transpose_pack(tt_hbm, out_hbm, ib0, ib1, tail_buf, ob0, ob1,
                    is0, is1, os0, os1):
    _tp_kernel(tt_hbm, out_hbm, [ib0, ib1], tail_buf, [ob0, ob1],
               [is0, is1], [os0, os1])


def _pool_kernel(x_hbm, table_hbm, out_hbm, idx_v, bufs, out_v, sems):
    wid = lax.axis_index("s") * NC + lax.axis_index("c")
    base = wid * BPW
    pltpu.sync_copy(x_hbm.at[pl.ds(base, BPW)], idx_v)

    himask = jnp.full((LANES,), 0xFFFF0000, dtype=jnp.uint32)
    sixteen = jnp.full((LANES,), 16, dtype=jnp.uint32)

    for k in range(NBUF):
        pltpu.async_copy(table_hbm.at[idx_v.at[k]], bufs[k], sems[k])

    def body(g, carry):
        for k in range(NBUF):
            b = g * NBUF + k
            buf = bufs[k]
            pltpu.make_async_copy(table_hbm.at[idx_v.at[b]], buf, sems[k]
                                  ).wait()
            accs = [jnp.zeros((LANES,), jnp.float32) for _ in range(4)]
            for l in range(L):
                for h in range(2):
                    v = buf[l, pl.ds(h * 16, 16)]
                    lo = plsc.bitcast(v << sixteen, jnp.float32)
                    hi = plsc.bitcast(v & himask, jnp.float32)
                    accs[2 * h] = accs[2 * h] + lo
                    accs[2 * h + 1] = accs[2 * h + 1] + hi

            @pl.when(b + NBUF < BPW)
            def _():
                pltpu.async_copy(table_hbm.at[idx_v.at[b + NBUF]], buf,
                                 sems[k])

            for c in range(4):
                out_v[b, pl.ds(c * LANES, LANES)] = accs[c]
        return carry

    lax.fori_loop(0, BPW // NBUF, body, 0)
    pltpu.sync_copy(out_v, out_hbm.at[pl.ds(base, BPW)])


@functools.partial(
    pl.kernel,
    mesh=plsc.VectorSubcoreMesh(core_axis_name="c", subcore_axis_name="s"),
    out_type=jax.ShapeDtypeStruct((B, EMB), jnp.float32),
    scratch_types=[
        pltpu.VMEM((BPW, L), jnp.int32),
        *[pltpu.VMEM((L, U32C), jnp.uint32) for _ in range(NBUF)],
        pltpu.VMEM((BPW, EMB), jnp.float32),
        *[pltpu.SemaphoreType.DMA for _ in range(NBUF)],
    ],
    compiler_params=pltpu.CompilerParams(use_tc_tiling_on_sc=False,
                                         needs_layout_passes=False),
)
def _pool(x_hbm, table_hbm, out_hbm, idx_v, *rest):
    bufs = list(rest[:NBUF])
    out_v = rest[NBUF]
    sems = list(rest[NBUF + 1:NBUF + 1 + NBUF])
    _pool_kernel(x_hbm, table_hbm, out_hbm, idx_v, bufs, out_v, sems)


def _mlp_kernel(h_ref, w1_ref, b1_ref, w2_ref, b2_ref, w3_ref, b3_ref, o_ref):
    dn = (((1,), (1,)), ((), ()))
    h = h_ref[...]
    z = lax.dot_general(h, w1_ref[...], dn, preferred_element_type=jnp.float32)
    z = jnp.maximum(z + b1_ref[...], 0.0)
    z = lax.dot_general(z, w2_ref[...], dn, preferred_element_type=jnp.float32)
    z = jnp.maximum(z + b2_ref[...], 0.0)
    z = lax.dot_general(z, w3_ref[...], dn, preferred_element_type=jnp.float32)
    o_ref[...] = z + b3_ref[...]


def kernel(x, table, W1, b1, W2, b2, W3, b3):
    tpk = _transpose_pack(table.T)
    h = _pool(x, tpk.reshape(VOCAB, U32C))
    w1p = W1 * jnp.float32(1.0 / L)
    return pl.pallas_call(
        _mlp_kernel,
        out_shape=jax.ShapeDtypeStruct((B, EMB), jnp.float32),
    )(h, w1p, b1.reshape(1, -1), W2, b2.reshape(1, -1), W3, b3.reshape(1, -1))


# speedup vs baseline: 1.4313x; 1.4313x over previous
import functools

import jax
import jax.numpy as jnp
import numpy as np
from jax import lax
from jax.experimental import pallas as pl
from jax.experimental.pallas import tpu as pltpu
from jax.experimental.pallas import tpu_sc as plsc

B = 4096
L = 50
EMB = 64
VOCAB = 100000
NC = 2
NS = 16
NW = NC * NS
BPW = B // NW
LANES = 16
NBUF = 4

CV = 128
NCHUNK = VOCAB // CV
TAIL = VOCAB - NCHUNK * CV
TAILW = NCHUNK % NW
U32C = EMB // 2
CHUNK_OUT = CV * U32C


MIDP = U32C + 1


def _tp_kernel(tt_hbm, out_hbm, in_bufs, tail_buf, mid_v, out_bufs, in_sems,
               out_sems):
    wid = lax.axis_index("s") * NC + lax.axis_index("c")
    iota = lax.iota(jnp.int32, LANES)
    iota_mid = iota * MIDP

    def start_in(i, k):
        c = wid + NW * i

        @pl.when(c < NCHUNK)
        def _():
            off = pl.multiple_of(c * CV, CV)
            pltpu.async_copy(tt_hbm.at[:, pl.ds(off, CV)], in_bufs[k],
                             in_sems[k])

    def transpose_block(src, nvoc, out_buf):
        def cbody(c, carry):
            base = iota_mid + c * (16 * MIDP)
            for d in range(EMB):
                uc = d % U32C
                v = plsc.bitcast(src[d, pl.ds(c * 16, 16)], jnp.int32)
                if d < U32C:
                    lo = lax.shift_right_logical(v, 16)
                    plsc.store_scatter(mid_v, [base + uc], lo)
                else:
                    hi = v & jnp.int32(-65536)
                    plsc.addupdate_scatter(mid_v, [base + uc], hi)
            return carry

        lax.fori_loop(0, nvoc // 16, cbody, 0)

        def vbody(v, carry):
            r0 = plsc.load_gather(mid_v, [iota + v * MIDP])
            r1 = plsc.load_gather(mid_v, [iota + (v * MIDP + 16)])
            out_buf[pl.ds(pl.multiple_of(v * U32C, 16), LANES)] = (
                plsc.bitcast(r0, jnp.uint32))
            out_buf[pl.ds(pl.multiple_of(v * U32C + 16, 16), LANES)] = (
                plsc.bitcast(r1, jnp.uint32))
            return carry

        lax.fori_loop(0, nvoc, vbody, 0)

    for k in range(2):
        start_in(k, k)

    def body(g, carry):
        for k in range(2):
            i = 2 * g + k
            c = wid + NW * i

            @pl.when(c < NCHUNK)
            def _():
                off = pl.multiple_of(c * CV, CV)
                pltpu.make_async_copy(tt_hbm.at[:, pl.ds(off, CV)],
                                      in_bufs[k], in_sems[k]).wait()

                @pl.when(i >= 2)
                def _():
                    pltpu.make_async_copy(
                        out_bufs[k], out_hbm.at[pl.ds(0, CHUNK_OUT)],
                        out_sems[k]).wait()

                transpose_block(in_bufs[k], CV, out_bufs[k])
                pltpu.async_copy(
                    out_bufs[k],
                    out_hbm.at[pl.ds(pl.multiple_of(c * CHUNK_OUT, 8),
                                     CHUNK_OUT)],
                    out_sems[k])
                start_in(i + 2, k)
        return carry

    niter = (NCHUNK + NW - 1) // NW
    lax.fori_loop(0, (niter + 1) // 2, body, 0)

    for k in range(2):
        pltpu.make_async_copy(out_bufs[k], out_hbm.at[pl.ds(0, CHUNK_OUT)],
                              out_sems[k]).wait()

    @pl.when(wid == TAILW)
    def _():
        pltpu.sync_copy(tt_hbm.at[:, pl.ds(NCHUNK * CV, TAIL)], tail_buf)
        transpose_block(tail_buf, TAIL, out_bufs[0])
        pltpu.sync_copy(out_bufs[0].at[pl.ds(0, TAIL * U32C)],
                        out_hbm.at[pl.ds(NCHUNK * CHUNK_OUT, TAIL * U32C)])


@functools.partial(
    pl.kernel,
    mesh=plsc.VectorSubcoreMesh(core_axis_name="c", subcore_axis_name="s"),
    out_type=jax.ShapeDtypeStruct((VOCAB * U32C,), jnp.uint32),
    scratch_types=[
        *[pltpu.VMEM((EMB, CV), jnp.float32) for _ in range(2)],
        pltpu.VMEM((EMB, TAIL), jnp.float32),
        pltpu.VMEM((CV * MIDP,), jnp.int32),
        *[pltpu.VMEM((CHUNK_OUT,), jnp.uint32) for _ in range(2)],
        *[pltpu.SemaphoreType.DMA for _ in range(4)],
    ],
    compiler_params=pltpu.CompilerParams(use_tc_tiling_on_sc=True,
                                         needs_layout_passes=False),
)
def _transpose_pack(tt_hbm, out_hbm, ib0, ib1, tail_buf, mid_v, ob0, ob1,
                    is0, is1, os0, os1):
    _tp_kernel(tt_hbm, out_hbm, [ib0, ib1], tail_buf, mid_v, [ob0, ob1],
               [is0, is1], [os0, os1])


def _pool_kernel(x_hbm, table_hbm, out_hbm, idx_v, bufs, out_v, sems):
    wid = lax.axis_index("s") * NC + lax.axis_index("c")
    base = wid * BPW
    pltpu.sync_copy(x_hbm.at[pl.ds(base, BPW)], idx_v)

    himask = jnp.full((LANES,), 0xFFFF0000, dtype=jnp.uint32)
    sixteen = jnp.full((LANES,), 16, dtype=jnp.uint32)

    for k in range(NBUF):
        pltpu.async_copy(table_hbm.at[idx_v.at[k]], bufs[k], sems[k])

    def body(g, carry):
        for k in range(NBUF):
            b = g * NBUF + k
            buf = bufs[k]
            pltpu.make_async_copy(table_hbm.at[idx_v.at[b]], buf, sems[k]
                                  ).wait()
            accs = [jnp.zeros((LANES,), jnp.float32) for _ in range(4)]
            for l in range(L):
                for h in range(2):
                    v = buf[l, pl.ds(h * 16, 16)]
                    lo = plsc.bitcast(v << sixteen, jnp.float32)
                    hi = plsc.bitcast(v & himask, jnp.float32)
                    accs[2 * h] = accs[2 * h] + lo
                    accs[2 * h + 1] = accs[2 * h + 1] + hi

            @pl.when(b + NBUF < BPW)
            def _():
                pltpu.async_copy(table_hbm.at[idx_v.at[b + NBUF]], buf,
                                 sems[k])

            for c in range(4):
                out_v[b, pl.ds(c * LANES, LANES)] = accs[c]
        return carry

    lax.fori_loop(0, BPW // NBUF, body, 0)
    pltpu.sync_copy(out_v, out_hbm.at[pl.ds(base, BPW)])


@functools.partial(
    pl.kernel,
    mesh=plsc.VectorSubcoreMesh(core_axis_name="c", subcore_axis_name="s"),
    out_type=jax.ShapeDtypeStruct((B, EMB), jnp.float32),
    scratch_types=[
        pltpu.VMEM((BPW, L), jnp.int32),
        *[pltpu.VMEM((L, U32C), jnp.uint32) for _ in range(NBUF)],
        pltpu.VMEM((BPW, EMB), jnp.float32),
        *[pltpu.SemaphoreType.DMA for _ in range(NBUF)],
    ],
    compiler_params=pltpu.CompilerParams(use_tc_tiling_on_sc=False,
                                         needs_layout_passes=False),
)
def _pool(x_hbm, table_hbm, out_hbm, idx_v, *rest):
    bufs = list(rest[:NBUF])
    out_v = rest[NBUF]
    sems = list(rest[NBUF + 1:NBUF + 1 + NBUF])
    _pool_kernel(x_hbm, table_hbm, out_hbm, idx_v, bufs, out_v, sems)


def _mlp_kernel(h_ref, w1_ref, b1_ref, w2_ref, b2_ref, w3_ref, b3_ref, o_ref):
    dn = (((1,), (1,)), ((), ()))
    h = h_ref[...]
    z = lax.dot_general(h, w1_ref[...], dn, preferred_element_type=jnp.float32)
    z = jnp.maximum(z + b1_ref[...], 0.0)
    z = lax.dot_general(z, w2_ref[...], dn, preferred_element_type=jnp.float32)
    z = jnp.maximum(z + b2_ref[...], 0.0)
    z = lax.dot_general(z, w3_ref[...], dn, preferred_element_type=jnp.float32)
    o_ref[...] = z + b3_ref[...]


_PERM = np.concatenate([np.arange(0, 16), np.arange(32, 48),
                        np.arange(16, 32), np.arange(48, 64)])


def kernel(x, table, W1, b1, W2, b2, W3, b3):
    tpk = _transpose_pack(table.T)
    h = _pool(x, tpk.reshape(VOCAB, U32C))
    w1p = W1[:, _PERM] * jnp.float32(1.0 / L)
    return pl.pallas_call(
        _mlp_kernel,
        out_shape=jax.ShapeDtypeStruct((B, EMB), jnp.float32),
    )(h, w1p, b1.reshape(1, -1), W2, b2.reshape(1, -1), W3, b3.reshape(1, -1))
